# Initial kernel scaffold; baseline (speedup 1.0000x reference)
import jax
import jax.numpy as jnp
from jax.experimental import pallas as pl


def kernel(s_self_feat, s_one_hop_feat, s_two_hop_feat, t_self_feat, t_one_hop_feat, t_two_hop_feat, neg_self_feat, neg_one_hop_feat, neg_two_hop_feat, s_his_time, s_his_his_time, t_his_time, t_his_his_time, neg_his_time, neg_his_his_time, s_edge_rate, s_t_rate, s_n_rate, W0, W1, W2, lam, training):
    raise NotImplementedError("write your pallas kernel here")


# speedup vs baseline: 1.0398x; 1.0398x over previous
import jax
import jax.numpy as jnp
from jax.experimental import pallas as pl
from jax.experimental.pallas import tpu as pltpu


def _branch_emb(self_ref, one_ref, two_ref, t1_ref, t2_ref, W0, W1, W2, lam):
    BB, H, _, D = two_ref.shape
    HID = W0.shape[1]
    t2 = t2_ref[...]
    a2 = jax.nn.softmax(-lam * t2, axis=-1)
    two = two_ref[...]
    red = jnp.sum(a2[..., None] * two, axis=2)
    m2 = jnp.dot(red.reshape(BB * H, D), W2, preferred_element_type=jnp.float32)
    one2 = one_ref[...].reshape(BB * H, D)
    h1 = jnp.tanh(jnp.dot(one2, W1, preferred_element_type=jnp.float32) + m2)
    t1 = t1_ref[...]
    a1 = jax.nn.softmax(-lam * t1, axis=-1)
    m1 = jnp.sum(a1[..., None] * h1.reshape(BB, H, HID), axis=1)
    emb = jnp.tanh(jnp.dot(self_ref[...], W0, preferred_element_type=jnp.float32) + m1)
    return emb


def _cos_rows(a, b):
    num = jnp.sum(a * b, axis=-1, keepdims=True)
    na = jnp.sqrt(jnp.sum(a * a, axis=-1, keepdims=True))
    nb = jnp.sqrt(jnp.sum(b * b, axis=-1, keepdims=True))
    return num / jnp.maximum(na * nb, 1e-8)


def _body(lam_ref, str_ref, snr_ref, W0_ref, W1_ref, W2_ref,
          s_self, s_one, s_two, s_t1, s_t2,
          t_self, t_one, t_two, t_t1, t_t2,
          n_self, n_one, n_two, n_t1, n_t2,
          s_out, t_out, n_out, L_out, acc_ref, *, total_b):
    lam = lam_ref[0, 0]
    W0 = W0_ref[...]
    W1 = W1_ref[...]
    W2 = W2_ref[...]

    s_emb = _branch_emb(s_self, s_one, s_two, s_t1, s_t2, W0, W1, W2, lam)
    t_emb = _branch_emb(t_self, t_one, t_two, t_t1, t_t2, W0, W1, W2, lam)
    n_emb = _branch_emb(n_self, n_one, n_two, n_t1, n_t2, W0, W1, W2, lam)

    s_out[...] = s_emb
    t_out[...] = t_emb
    n_out[...] = n_emb

    pos = (1.0 - _cos_rows(s_emb, t_emb)) * str_ref[...]
    cn = _cos_rows(s_emb, n_emb)
    eterm = jnp.where(snr_ref[...] > 0.0, 1.0 - cn, jnp.maximum(cn, 0.0))

    @pl.when(pl.program_id(0) == 0)
    def _init():
        acc_ref[0] = 0.0
        acc_ref[1] = 0.0

    acc_ref[0] += jnp.sum(pos)
    acc_ref[1] += jnp.sum(eterm)

    @pl.when(pl.program_id(0) == pl.num_programs(0) - 1)
    def _fin():
        lpos = acc_ref[0] / total_b
        el = acc_ref[1] / total_b
        L_out[...] = jnp.full((1, 1), lpos + el * lpos, dtype=jnp.float32)


def kernel(s_self_feat, s_one_hop_feat, s_two_hop_feat,
           t_self_feat, t_one_hop_feat, t_two_hop_feat,
           neg_self_feat, neg_one_hop_feat, neg_two_hop_feat,
           s_his_time, s_his_his_time, t_his_time, t_his_his_time,
           neg_his_time, neg_his_his_time,
           s_edge_rate, s_t_rate, s_n_rate, W0, W1, W2, lam, training=False):
    B, H, D = s_one_hop_feat.shape
    HID = W0.shape[1]
    BB = 16
    grid = (B // BB,)

    lam2 = jnp.reshape(lam, (1, 1))
    str2 = jnp.reshape(s_t_rate, (B, 1))
    snr2 = jnp.reshape(s_n_rate, (B, 1))

    spec_w = pl.BlockSpec((D, HID), lambda i: (0, 0))
    spec_self = pl.BlockSpec((BB, D), lambda i: (i, 0))
    spec_one = pl.BlockSpec((BB, H, D), lambda i: (i, 0, 0))
    spec_two = pl.BlockSpec((BB, H, H, D), lambda i: (i, 0, 0, 0))
    spec_t1 = pl.BlockSpec((BB, H), lambda i: (i, 0))
    spec_t2 = pl.BlockSpec((BB, H, H), lambda i: (i, 0, 0))
    spec_rate = pl.BlockSpec((BB, 1), lambda i: (i, 0))

    import functools
    body = functools.partial(_body, total_b=float(B))

    outs = pl.pallas_call(
        body,
        grid=grid,
        in_specs=[
            pl.BlockSpec(memory_space=pltpu.SMEM),
            spec_rate, spec_rate,
            spec_w, spec_w, spec_w,
            spec_self, spec_one, spec_two, spec_t1, spec_t2,
            spec_self, spec_one, spec_two, spec_t1, spec_t2,
            spec_self, spec_one, spec_two, spec_t1, spec_t2,
        ],
        out_specs=[
            pl.BlockSpec((BB, HID), lambda i: (i, 0)),
            pl.BlockSpec((BB, HID), lambda i: (i, 0)),
            pl.BlockSpec((BB, HID), lambda i: (i, 0)),
            pl.BlockSpec((1, 1), lambda i: (0, 0)),
        ],
        out_shape=[
            jax.ShapeDtypeStruct((B, HID), jnp.float32),
            jax.ShapeDtypeStruct((B, HID), jnp.float32),
            jax.ShapeDtypeStruct((B, HID), jnp.float32),
            jax.ShapeDtypeStruct((1, 1), jnp.float32),
        ],
        scratch_shapes=[pltpu.SMEM((2,), jnp.float32)],
    )(lam2, str2, snr2, W0, W1, W2,
      s_self_feat, s_one_hop_feat, s_two_hop_feat, s_his_time, s_his_his_time,
      t_self_feat, t_one_hop_feat, t_two_hop_feat, t_his_time, t_his_his_time,
      neg_self_feat, neg_one_hop_feat, neg_two_hop_feat, neg_his_time, neg_his_his_time)

    s_emb, t_emb, n_emb, L = outs
    L0 = L[0, 0]
    return (L0, s_emb, t_emb, s_emb, n_emb)


# speedup vs baseline: 1.2479x; 1.2002x over previous
import jax
import jax.numpy as jnp
from jax.experimental import pallas as pl
from jax.experimental.pallas import tpu as pltpu


def _branch_emb(self_ref, one_ref, two_ref, t1_ref, t2_ref, W0, W1, W2, lam):
    BB, H, _, D = two_ref.shape
    HID = W0.shape[1]
    t2 = t2_ref[...]
    a2 = jax.nn.softmax(-lam * t2, axis=-1)
    two = two_ref[...]
    red = jnp.sum(a2[..., None] * two, axis=2)
    m2 = jnp.dot(red.reshape(BB * H, D), W2, preferred_element_type=jnp.float32)
    one2 = one_ref[...].reshape(BB * H, D)
    h1 = jnp.tanh(jnp.dot(one2, W1, preferred_element_type=jnp.float32) + m2)
    t1 = t1_ref[...]
    a1 = jax.nn.softmax(-lam * t1, axis=-1)
    m1 = jnp.sum(a1[..., None] * h1.reshape(BB, H, HID), axis=1)
    emb = jnp.tanh(jnp.dot(self_ref[...], W0, preferred_element_type=jnp.float32) + m1)
    return emb


def _cos_rows(a, b):
    num = jnp.sum(a * b, axis=-1, keepdims=True)
    na = jnp.sqrt(jnp.sum(a * a, axis=-1, keepdims=True))
    nb = jnp.sqrt(jnp.sum(b * b, axis=-1, keepdims=True))
    return num / jnp.maximum(na * nb, 1e-8)


def _body(lam_ref, str_ref, snr_ref, W0_ref, W1_ref, W2_ref,
          s_self, s_one, s_two, s_t1, s_t2,
          t_self, t_one, t_two, t_t1, t_t2,
          n_self, n_one, n_two, n_t1, n_t2,
          s_out, t_out, n_out, L_out, acc_ref, *, total_b):
    lam = lam_ref[0, 0]
    W0 = W0_ref[...]
    W1 = W1_ref[...]
    W2 = W2_ref[...]

    s_emb = _branch_emb(s_self, s_one, s_two, s_t1, s_t2, W0, W1, W2, lam)
    t_emb = _branch_emb(t_self, t_one, t_two, t_t1, t_t2, W0, W1, W2, lam)
    n_emb = _branch_emb(n_self, n_one, n_two, n_t1, n_t2, W0, W1, W2, lam)

    s_out[...] = s_emb
    t_out[...] = t_emb
    n_out[...] = n_emb

    pos = (1.0 - _cos_rows(s_emb, t_emb)) * str_ref[...]
    cn = _cos_rows(s_emb, n_emb)
    eterm = jnp.where(snr_ref[...] > 0.0, 1.0 - cn, jnp.maximum(cn, 0.0))

    @pl.when(pl.program_id(0) == 0)
    def _init():
        acc_ref[0] = 0.0
        acc_ref[1] = 0.0

    acc_ref[0] += jnp.sum(pos)
    acc_ref[1] += jnp.sum(eterm)

    @pl.when(pl.program_id(0) == pl.num_programs(0) - 1)
    def _fin():
        lpos = acc_ref[0] / total_b
        el = acc_ref[1] / total_b
        L_out[...] = jnp.full((1, 1), lpos + el * lpos, dtype=jnp.float32)


def kernel(s_self_feat, s_one_hop_feat, s_two_hop_feat,
           t_self_feat, t_one_hop_feat, t_two_hop_feat,
           neg_self_feat, neg_one_hop_feat, neg_two_hop_feat,
           s_his_time, s_his_his_time, t_his_time, t_his_his_time,
           neg_his_time, neg_his_his_time,
           s_edge_rate, s_t_rate, s_n_rate, W0, W1, W2, lam, training=False):
    B, H, D = s_one_hop_feat.shape
    HID = W0.shape[1]
    BB = 32
    grid = (B // BB,)

    lam2 = jnp.reshape(lam, (1, 1))
    str2 = jnp.reshape(s_t_rate, (B, 1))
    snr2 = jnp.reshape(s_n_rate, (B, 1))

    spec_w = pl.BlockSpec((D, HID), lambda i: (0, 0))
    spec_self = pl.BlockSpec((BB, D), lambda i: (i, 0))
    spec_one = pl.BlockSpec((BB, H, D), lambda i: (i, 0, 0))
    spec_two = pl.BlockSpec((BB, H, H, D), lambda i: (i, 0, 0, 0))
    spec_t1 = pl.BlockSpec((BB, H), lambda i: (i, 0))
    spec_t2 = pl.BlockSpec((BB, H, H), lambda i: (i, 0, 0))
    spec_rate = pl.BlockSpec((BB, 1), lambda i: (i, 0))

    import functools
    body = functools.partial(_body, total_b=float(B))

    outs = pl.pallas_call(
        body,
        grid=grid,
        in_specs=[
            pl.BlockSpec(memory_space=pltpu.SMEM),
            spec_rate, spec_rate,
            spec_w, spec_w, spec_w,
            spec_self, spec_one, spec_two, spec_t1, spec_t2,
            spec_self, spec_one, spec_two, spec_t1, spec_t2,
            spec_self, spec_one, spec_two, spec_t1, spec_t2,
        ],
        out_specs=[
            pl.BlockSpec((BB, HID), lambda i: (i, 0)),
            pl.BlockSpec((BB, HID), lambda i: (i, 0)),
            pl.BlockSpec((BB, HID), lambda i: (i, 0)),
            pl.BlockSpec((1, 1), lambda i: (0, 0)),
        ],
        out_shape=[
            jax.ShapeDtypeStruct((B, HID), jnp.float32),
            jax.ShapeDtypeStruct((B, HID), jnp.float32),
            jax.ShapeDtypeStruct((B, HID), jnp.float32),
            jax.ShapeDtypeStruct((1, 1), jnp.float32),
        ],
        scratch_shapes=[pltpu.SMEM((2,), jnp.float32)],
    )(lam2, str2, snr2, W0, W1, W2,
      s_self_feat, s_one_hop_feat, s_two_hop_feat, s_his_time, s_his_his_time,
      t_self_feat, t_one_hop_feat, t_two_hop_feat, t_his_time, t_his_his_time,
      neg_self_feat, neg_one_hop_feat, neg_two_hop_feat, neg_his_time, neg_his_his_time)

    s_emb, t_emb, n_emb, L = outs
    L0 = L[0, 0]
    return (L0, s_emb, t_emb, s_emb, n_emb)


# speedup vs baseline: 1.3423x; 1.0756x over previous
import jax
import jax.numpy as jnp
from jax.experimental import pallas as pl
from jax.experimental.pallas import tpu as pltpu


def _branch_emb(self_ref, one_ref, two_ref, t1_ref, t2_ref, W0, W1, W2, lam):
    BB, H, _, D = two_ref.shape
    HID = W0.shape[1]
    t2 = t2_ref[...]
    a2 = jax.nn.softmax(-lam * t2, axis=-1)
    two = two_ref[...]
    red = jnp.sum(a2[..., None] * two, axis=2)
    m2 = jnp.dot(red.reshape(BB * H, D), W2, preferred_element_type=jnp.float32)
    one2 = one_ref[...].reshape(BB * H, D)
    h1 = jnp.tanh(jnp.dot(one2, W1, preferred_element_type=jnp.float32) + m2)
    t1 = t1_ref[...]
    a1 = jax.nn.softmax(-lam * t1, axis=-1)
    m1 = jnp.sum(a1[..., None] * h1.reshape(BB, H, HID), axis=1)
    emb = jnp.tanh(jnp.dot(self_ref[...], W0, preferred_element_type=jnp.float32) + m1)
    return emb


def _cos_rows(a, b):
    num = jnp.sum(a * b, axis=-1, keepdims=True)
    na = jnp.sqrt(jnp.sum(a * a, axis=-1, keepdims=True))
    nb = jnp.sqrt(jnp.sum(b * b, axis=-1, keepdims=True))
    return num / jnp.maximum(na * nb, 1e-8)


def _body(lam_ref, str_ref, snr_ref, W0_ref, W1_ref, W2_ref,
          s_self, s_one, s_two, s_t1, s_t2,
          t_self, t_one, t_two, t_t1, t_t2,
          n_self, n_one, n_two, n_t1, n_t2,
          s_out, t_out, n_out, L_out, acc_ref, *, total_b):
    lam = lam_ref[0, 0]
    W0 = W0_ref[...]
    W1 = W1_ref[...]
    W2 = W2_ref[...]

    s_emb = _branch_emb(s_self, s_one, s_two, s_t1, s_t2, W0, W1, W2, lam)
    t_emb = _branch_emb(t_self, t_one, t_two, t_t1, t_t2, W0, W1, W2, lam)
    n_emb = _branch_emb(n_self, n_one, n_two, n_t1, n_t2, W0, W1, W2, lam)

    s_out[...] = s_emb
    t_out[...] = t_emb
    n_out[...] = n_emb

    pos = (1.0 - _cos_rows(s_emb, t_emb)) * str_ref[...]
    cn = _cos_rows(s_emb, n_emb)
    eterm = jnp.where(snr_ref[...] > 0.0, 1.0 - cn, jnp.maximum(cn, 0.0))

    @pl.when(pl.program_id(0) == 0)
    def _init():
        acc_ref[0] = 0.0
        acc_ref[1] = 0.0

    acc_ref[0] += jnp.sum(pos)
    acc_ref[1] += jnp.sum(eterm)

    @pl.when(pl.program_id(0) == pl.num_programs(0) - 1)
    def _fin():
        lpos = acc_ref[0] / total_b
        el = acc_ref[1] / total_b
        L_out[...] = jnp.full((1, 1), lpos + el * lpos, dtype=jnp.float32)


def kernel(s_self_feat, s_one_hop_feat, s_two_hop_feat,
           t_self_feat, t_one_hop_feat, t_two_hop_feat,
           neg_self_feat, neg_one_hop_feat, neg_two_hop_feat,
           s_his_time, s_his_his_time, t_his_time, t_his_his_time,
           neg_his_time, neg_his_his_time,
           s_edge_rate, s_t_rate, s_n_rate, W0, W1, W2, lam, training=False):
    B, H, D = s_one_hop_feat.shape
    HID = W0.shape[1]
    BB = 64
    grid = (B // BB,)

    lam2 = jnp.reshape(lam, (1, 1))
    str2 = jnp.reshape(s_t_rate, (B, 1))
    snr2 = jnp.reshape(s_n_rate, (B, 1))

    spec_w = pl.BlockSpec((D, HID), lambda i: (0, 0))
    spec_self = pl.BlockSpec((BB, D), lambda i: (i, 0))
    spec_one = pl.BlockSpec((BB, H, D), lambda i: (i, 0, 0))
    spec_two = pl.BlockSpec((BB, H, H, D), lambda i: (i, 0, 0, 0))
    spec_t1 = pl.BlockSpec((BB, H), lambda i: (i, 0))
    spec_t2 = pl.BlockSpec((BB, H, H), lambda i: (i, 0, 0))
    spec_rate = pl.BlockSpec((BB, 1), lambda i: (i, 0))

    import functools
    body = functools.partial(_body, total_b=float(B))

    outs = pl.pallas_call(
        body,
        grid=grid,
        in_specs=[
            pl.BlockSpec(memory_space=pltpu.SMEM),
            spec_rate, spec_rate,
            spec_w, spec_w, spec_w,
            spec_self, spec_one, spec_two, spec_t1, spec_t2,
            spec_self, spec_one, spec_two, spec_t1, spec_t2,
            spec_self, spec_one, spec_two, spec_t1, spec_t2,
        ],
        out_specs=[
            pl.BlockSpec((BB, HID), lambda i: (i, 0)),
            pl.BlockSpec((BB, HID), lambda i: (i, 0)),
            pl.BlockSpec((BB, HID), lambda i: (i, 0)),
            pl.BlockSpec((1, 1), lambda i: (0, 0)),
        ],
        out_shape=[
            jax.ShapeDtypeStruct((B, HID), jnp.float32),
            jax.ShapeDtypeStruct((B, HID), jnp.float32),
            jax.ShapeDtypeStruct((B, HID), jnp.float32),
            jax.ShapeDtypeStruct((1, 1), jnp.float32),
        ],
        scratch_shapes=[pltpu.SMEM((2,), jnp.float32)],
    )(lam2, str2, snr2, W0, W1, W2,
      s_self_feat, s_one_hop_feat, s_two_hop_feat, s_his_time, s_his_his_time,
      t_self_feat, t_one_hop_feat, t_two_hop_feat, t_his_time, t_his_his_time,
      neg_self_feat, neg_one_hop_feat, neg_two_hop_feat, neg_his_time, neg_his_his_time)

    s_emb, t_emb, n_emb, L = outs
    L0 = L[0, 0]
    return (L0, s_emb, t_emb, s_emb, n_emb)
